# baseline (speedup 1.0000x reference)
import functools
import math

import jax
import jax.numpy as jnp
from jax import lax
from jax.experimental import pallas as pl
from jax.experimental.pallas import tpu as pltpu
from jax.experimental.pallas import tpu_sc as plsc

EMB_SIZE = 64
SCALE = math.sqrt(EMB_SIZE)

NUM_CORES = 2
NUM_SUBCORES = 16
NUM_WORKERS = NUM_CORES * NUM_SUBCORES
LANES = 16

VOCAB_ROWS = 1000000
SEQ = 50
SEQ_PAD = 56
N_BATCH = 16384
BT_TOTAL = N_BATCH // 128
BT_PER_W = BT_TOTAL // NUM_WORKERS
UNITS = BT_PER_W * SEQ


def _emb_body(tok_hbm, tab_hbm, out_hbm, tok_v, rows_v, t_v,
              sem_i, sem_g0, sem_g1, sem_g2, sem_g3, sem_w0, sem_w1):
  sem_g = (sem_g0, sem_g1, sem_g2, sem_g3)
  sem_w = (sem_w0, sem_w1)
  wid = lax.axis_index("s") * NUM_CORES + lax.axis_index("c")
  bt0 = wid * BT_PER_W

  stage = [
      pltpu.make_async_copy(
          tok_hbm.at[st, bt0 + i], tok_v.at[st, i], sem_i
      )
      for st in range(7)
      for i in range(BT_PER_W)
  ]
  for d in stage:
    d.start()
  for d in stage:
    d.wait()

  iota = lax.iota(jnp.int32, LANES)
  rowv = [iota + (j * LANES) for j in range(8)]

  def gather_desc(u, g):
    bt_local = u // SEQ
    s = u % SEQ
    idx_ref = tok_v.at[s // 8, bt_local, s % 8]
    return pltpu.make_async_copy(tab_hbm.at[idx_ref], rows_v.at[g], sem_g[g])

  def write_descs(u, b):
    bt_local = u // SEQ
    s = u % SEQ
    return [
        pltpu.make_async_copy(
            t_v.at[b, pl.ds(dt * 8, 8), :],
            out_hbm.at[s, dt, bt0 + bt_local],
            sem_w[b],
        )
        for dt in range(8)
    ]

  gather_desc(0, 0).start()
  gather_desc(1, 1).start()

  @pl.loop(0, UNITS // 4)
  def _(u4):
    for ub in range(4):
      u = u4 * 4 + ub
      b = ub % 2

      @pl.when(u + 2 < UNITS)
      def _():
        gather_desc(u + 2, (ub + 2) % 4).start()

      gather_desc(u, ub).wait()

      @pl.when(u >= 2)
      def _():
        for d in write_descs(u, b):
          d.wait()

      @pl.loop(0, LANES, unroll=4)
      def _(k):
        perm = (iota + k) & (LANES - 1)
        for db in range(EMB_SIZE // LANES):
          col = perm + (db * LANES)
          for j in range(8):
            plsc.store_scatter(
                t_v.at[b],
                [col, rowv[j]],
                plsc.load_gather(rows_v.at[ub], [rowv[j], col]) * SCALE,
            )

      for d in write_descs(u, b):
        d.start()

  for b in range(2):
    for d in write_descs(UNITS - 2 + b, b):
      d.wait()


@jax.jit
def _emb_call(tok4d, tab_padded):
  mesh = plsc.VectorSubcoreMesh(
      core_axis_name="c", subcore_axis_name="s", num_cores=NUM_CORES
  )
  return pl.kernel(
      _emb_body,
      out_type=jax.ShapeDtypeStruct((SEQ, 8, 128, 8, 128), jnp.float32),
      mesh=mesh,
      scratch_types=[
          pltpu.VMEM((7, BT_PER_W, 8, 128), jnp.int32),
          pltpu.VMEM((4, 128, EMB_SIZE), jnp.float32),
          pltpu.VMEM((2, EMB_SIZE, 128), jnp.float32),
          pltpu.SemaphoreType.DMA,
          pltpu.SemaphoreType.DMA,
          pltpu.SemaphoreType.DMA,
          pltpu.SemaphoreType.DMA,
          pltpu.SemaphoreType.DMA,
          pltpu.SemaphoreType.DMA,
          pltpu.SemaphoreType.DMA,
      ],
      compiler_params=pltpu.CompilerParams(use_tc_tiling_on_sc=False, needs_layout_passes=False),
  )(tok4d, tab_padded)


def kernel(tokens, embedding_weight):
  n_batch, seq = tokens.shape
  assert (n_batch, seq) == (N_BATCH, SEQ)
  tok_t = jnp.pad((tokens.astype(jnp.int32) * 2).T,
                  ((0, SEQ_PAD - SEQ), (0, 0)))
  tok4d = tok_t.reshape(7, 8, 128, 128).transpose(0, 2, 1, 3)
  tab2m = jnp.pad(
      embedding_weight, ((0, 0), (0, 128 - EMB_SIZE))
  ).reshape(2 * VOCAB_ROWS, EMB_SIZE)
  out5 = _emb_call(tok4d, tab2m)
  return out5.transpose(2, 4, 0, 1, 3).reshape(n_batch, seq, EMB_SIZE)


# speedup vs baseline: 1.0337x; 1.0337x over previous
import functools
import math

import jax
import jax.numpy as jnp
from jax import lax
from jax.experimental import pallas as pl
from jax.experimental.pallas import tpu as pltpu
from jax.experimental.pallas import tpu_sc as plsc

EMB_SIZE = 64
SCALE = math.sqrt(EMB_SIZE)

NUM_CORES = 2
NUM_SUBCORES = 16
NUM_WORKERS = NUM_CORES * NUM_SUBCORES
LANES = 16

VOCAB_ROWS = 1000000
SEQ = 50
SEQ_PAD = 56
N_BATCH = 16384
BT_TOTAL = N_BATCH // 128
BT_PER_W = BT_TOTAL // NUM_WORKERS
UNITS = BT_PER_W * SEQ


def _emb_body(tok_hbm, tab_hbm, out_hbm, tok_v, rows_v, t_v,
              sem_i, sem_g0, sem_g1, sem_g2, sem_g3,
              sem_w0, sem_w1, sem_w2, sem_w3):
  sem_g = (sem_g0, sem_g1, sem_g2, sem_g3)
  sem_w = (sem_w0, sem_w1, sem_w2, sem_w3)
  wid = lax.axis_index("s") * NUM_CORES + lax.axis_index("c")
  bt0 = wid * BT_PER_W

  stage = [
      pltpu.make_async_copy(
          tok_hbm.at[st, bt0 + i], tok_v.at[st, i], sem_i
      )
      for st in range(7)
      for i in range(BT_PER_W)
  ]
  for d in stage:
    d.start()
  for d in stage:
    d.wait()

  iota = lax.iota(jnp.int32, LANES)
  rowv = [iota + (j * LANES) for j in range(8)]

  def gather_desc(u, g):
    bt_local = u // SEQ
    s = u % SEQ
    idx_ref = tok_v.at[s // 8, bt_local, s % 8]
    return pltpu.make_async_copy(tab_hbm.at[idx_ref], rows_v.at[g], sem_g[g])

  def write_descs(u, b):
    bt_local = u // SEQ
    s = u % SEQ
    return [
        pltpu.make_async_copy(
            t_v.at[b, pl.ds(dt * 8, 8), :],
            out_hbm.at[s, dt, bt0 + bt_local],
            sem_w[b],
        )
        for dt in range(8)
    ]

  gather_desc(0, 0).start()
  gather_desc(1, 1).start()

  @pl.loop(0, UNITS // 4)
  def _(u4):
    for ub in range(4):
      u = u4 * 4 + ub
      b = ub % 2

      @pl.when(u + 2 < UNITS)
      def _():
        gather_desc(u + 2, (ub + 2) % 4).start()

      gather_desc(u, ub).wait()

      @pl.when(u >= 4)
      def _():
        for d in write_descs(u, ub):
          d.wait()

      @pl.loop(0, LANES, unroll=2)
      def _(k):
        perm = (iota + k) & (LANES - 1)
        for db in range(EMB_SIZE // LANES):
          col = perm + (db * LANES)
          for j in range(8):
            plsc.store_scatter(
                t_v.at[ub],
                [col, rowv[j]],
                plsc.load_gather(rows_v.at[ub], [rowv[j], col]) * SCALE,
            )

      for d in write_descs(u, ub):
        d.start()

  for q in range(4):
    for d in write_descs(UNITS - 4 + q, q):
      d.wait()


@jax.jit
def _emb_call(tok4d, tab_padded):
  mesh = plsc.VectorSubcoreMesh(
      core_axis_name="c", subcore_axis_name="s", num_cores=NUM_CORES
  )
  return pl.kernel(
      _emb_body,
      out_type=jax.ShapeDtypeStruct((SEQ, 8, 128, 8, 128), jnp.float32),
      mesh=mesh,
      scratch_types=[
          pltpu.VMEM((7, BT_PER_W, 8, 128), jnp.int32),
          pltpu.VMEM((4, 128, EMB_SIZE), jnp.float32),
          pltpu.VMEM((4, EMB_SIZE, 128), jnp.float32),
          pltpu.SemaphoreType.DMA,
          pltpu.SemaphoreType.DMA,
          pltpu.SemaphoreType.DMA,
          pltpu.SemaphoreType.DMA,
          pltpu.SemaphoreType.DMA,
          pltpu.SemaphoreType.DMA,
          pltpu.SemaphoreType.DMA,
          pltpu.SemaphoreType.DMA,
          pltpu.SemaphoreType.DMA,
      ],
      compiler_params=pltpu.CompilerParams(use_tc_tiling_on_sc=False, needs_layout_passes=False),
  )(tok4d, tab_padded)


def kernel(tokens, embedding_weight):
  n_batch, seq = tokens.shape
  assert (n_batch, seq) == (N_BATCH, SEQ)
  tok_t = jnp.pad((tokens.astype(jnp.int32) * 2).T,
                  ((0, SEQ_PAD - SEQ), (0, 0)))
  tok4d = tok_t.reshape(7, 8, 128, 128).transpose(0, 2, 1, 3)
  tab2m = jnp.pad(
      embedding_weight, ((0, 0), (0, 128 - EMB_SIZE))
  ).reshape(2 * VOCAB_ROWS, EMB_SIZE)
  out5 = _emb_call(tok4d, tab2m)
  return out5.transpose(2, 4, 0, 1, 3).reshape(n_batch, seq, EMB_SIZE)
